# baseline (speedup 1.0000x reference)
import numpy as np
import jax
import jax.numpy as jnp
from jax.experimental import pallas as pl
from jax.experimental.pallas import tpu as pltpu

_B, _H, _S, _DH, _BLK = 2, 16, 4096, 64, 64
_NB = _S // _BLK
_R = 3
_M = _NB - 2
_VP = 128
_SCALE = 1.0 / np.sqrt(_DH)
_BOUND_PAD = 1.01


def _dot_t(a, b):
    return jax.lax.dot_general(
        a, b, (((1,), (1,)), ((), ())), preferred_element_type=jnp.float32
    )


def _dot(a, b):
    return jnp.dot(a, b, preferred_element_type=jnp.float32)


def _bigbird_kernel(rand_ref, q_ref, k_ref, v_ref, o_ref,
                    kb_ref, vp_ref, kg_ref, vg_ref):
    h = pl.program_id(1)

    kb_ref[...] = k_ref[0, 0].astype(jnp.bfloat16)
    vp_ref[:, 0:_DH] = v_ref[0, 0].astype(jnp.bfloat16)
    vp_ref[:, _DH:_VP] = jnp.zeros((_S, _VP - _DH), jnp.bfloat16)
    vp_ref[:, _DH:_DH + 1] = jnp.ones((_S, 1), jnp.bfloat16)
    kg_ref[0:_BLK] = kb_ref[0:_BLK]
    kg_ref[_BLK:2 * _BLK] = kb_ref[_S - _BLK:_S]
    vg_ref[0:_BLK] = vp_ref[0:_BLK]
    vg_ref[_BLK:2 * _BLK] = vp_ref[_S - _BLK:_S]

    kf = k_ref[0, 0]
    kmax = jnp.sqrt(jnp.max(jnp.sum(kf * kf, axis=-1))) * _BOUND_PAD

    def row_bound(q_rows):
        qf = q_rows.astype(jnp.float32)
        return jnp.sqrt(
            jnp.sum(qf * qf, axis=-1, keepdims=True)
        ) * (kmax * _BOUND_PAD)

    qg = jnp.concatenate(
        [q_ref[0, 0, 0:_BLK], q_ref[0, 0, _S - _BLK:_S]], axis=0
    )
    qg = (qg * _SCALE).astype(jnp.bfloat16)
    mx_g = row_bound(qg)
    s = _dot_t(qg, kb_ref[...])
    p = jnp.exp(s - mx_g).astype(jnp.bfloat16)
    res = _dot(p, vp_ref[...])
    og = res[:, 0:_DH] / res[:, _DH:_DH + 1]
    o_ref[0, 0, 0:_BLK] = og[0:_BLK]
    o_ref[0, 0, _S - _BLK:_S] = og[_BLK:]

    def qk_phase(m):
        r0 = rand_ref[h, m, 0]
        r1 = rand_ref[h, m, 1]
        r2 = rand_ref[h, m, 2]
        qm = q_ref[0, 0, pl.ds((m + 1) * _BLK, _BLK)]
        qm = (qm * _SCALE).astype(jnp.bfloat16)
        mx = row_bound(qm)
        s_g = _dot_t(qm, kg_ref[...])
        s_w = _dot_t(qm, kb_ref[pl.ds(m * _BLK, 3 * _BLK)])
        s_0 = _dot_t(qm, kb_ref[pl.ds(r0 * _BLK, _BLK)])
        s_1 = _dot_t(qm, kb_ref[pl.ds(r1 * _BLK, _BLK)])
        s_2 = _dot_t(qm, kb_ref[pl.ds(r2 * _BLK, _BLK)])
        es = [jnp.exp(sp - mx).astype(jnp.bfloat16)
              for sp in (s_g, s_w, s_0, s_1, s_2)]
        return (es, r0, r1, r2)

    def pv_phase(m, state):
        es, r0, r1, r2 = state
        acc = _dot(es[0], vg_ref[...])
        acc = acc + _dot(es[1], vp_ref[pl.ds(m * _BLK, 3 * _BLK)])
        acc = acc + _dot(es[2], vp_ref[pl.ds(r0 * _BLK, _BLK)])
        acc = acc + _dot(es[3], vp_ref[pl.ds(r1 * _BLK, _BLK)])
        acc = acc + _dot(es[4], vp_ref[pl.ds(r2 * _BLK, _BLK)])
        o_ref[0, 0, pl.ds((m + 1) * _BLK, _BLK)] = (
            acc[:, 0:_DH] / acc[:, _DH:_DH + 1]
        )

    _G = 6

    def qk_group(i):
        return [qk_phase(i * _G + j) for j in range(_G)]

    def pv_group(i, states):
        for j, st in enumerate(states):
            pv_phase(i * _G + j, st)

    def body(i, carry):
        nxt = qk_group(i)
        pv_group(i - 1, carry)
        return nxt

    last = jax.lax.fori_loop(1, 10, body, qk_group(0), unroll=False)
    pv_group(9, last)
    states = [qk_phase(m) for m in (60, 61)]
    for m, st in zip((60, 61), states):
        pv_phase(m, st)


def kernel(q, k, v, rand_attn):
    rand = rand_attn.astype(jnp.int32)

    def _spec(b, h, rand_ref):
        return (b, h, 0, 0)

    qkv_spec = pl.BlockSpec((1, 1, _S, _DH), _spec)
    out = pl.pallas_call(
        _bigbird_kernel,
        grid_spec=pltpu.PrefetchScalarGridSpec(
            num_scalar_prefetch=1,
            grid=(_B, _H),
            in_specs=[qkv_spec, qkv_spec, qkv_spec],
            out_specs=qkv_spec,
            scratch_shapes=[
                pltpu.VMEM((_S, _DH), jnp.bfloat16),
                pltpu.VMEM((_S, _VP), jnp.bfloat16),
                pltpu.VMEM((2 * _BLK, _DH), jnp.bfloat16),
                pltpu.VMEM((2 * _BLK, _VP), jnp.bfloat16),
            ],
        ),
        out_shape=jax.ShapeDtypeStruct((_B, _H, _S, _DH), jnp.float32),
        compiler_params=pltpu.CompilerParams(
            dimension_semantics=("parallel", "parallel"),
        ),
    )(rand, q, k, v)
    return out


# speedup vs baseline: 1.1196x; 1.1196x over previous
import numpy as np
import jax
import jax.numpy as jnp
from jax.experimental import pallas as pl
from jax.experimental.pallas import tpu as pltpu

_B, _H, _S, _DH, _BLK = 2, 16, 4096, 64, 64
_NB = _S // _BLK
_R = 3
_M = _NB - 2
_VP = 128
_SCALE = 1.0 / np.sqrt(_DH)
_BOUND_PAD = 1.01


def _dot_t(a, b):
    return jax.lax.dot_general(
        a, b, (((1,), (1,)), ((), ())), preferred_element_type=jnp.float32
    )


def _dot(a, b):
    return jnp.dot(a, b, preferred_element_type=jnp.float32)


def _bigbird_kernel(rand_ref, q_ref, k_ref, v_ref, o_ref,
                    kb_ref, vp_ref, kg_ref, vg_ref):
    h = pl.program_id(1)

    kb_ref[...] = k_ref[0, 0].astype(jnp.bfloat16)
    vp_ref[:, 0:_DH] = v_ref[0, 0].astype(jnp.bfloat16)
    vp_ref[:, _DH:_VP] = jnp.zeros((_S, _VP - _DH), jnp.bfloat16)
    vp_ref[:, _DH:_DH + 1] = jnp.ones((_S, 1), jnp.bfloat16)
    kg_ref[0:_BLK] = kb_ref[0:_BLK]
    kg_ref[_BLK:2 * _BLK] = kb_ref[_S - _BLK:_S]
    vg_ref[0:_BLK] = vp_ref[0:_BLK]
    vg_ref[_BLK:2 * _BLK] = vp_ref[_S - _BLK:_S]

    kf = k_ref[0, 0]
    kmax = jnp.sqrt(jnp.max(jnp.sum(kf * kf, axis=-1))) * _BOUND_PAD

    def row_bound(q_rows):
        qf = q_rows.astype(jnp.float32)
        return jnp.sqrt(
            jnp.sum(qf * qf, axis=-1, keepdims=True)
        ) * (kmax * _BOUND_PAD)

    qg = jnp.concatenate(
        [q_ref[0, 0, 0:_BLK], q_ref[0, 0, _S - _BLK:_S]], axis=0
    )
    qg = (qg * _SCALE).astype(jnp.bfloat16)
    mx_g = row_bound(qg)
    s = _dot_t(qg, kb_ref[...])
    p = jnp.exp(s - mx_g).astype(jnp.bfloat16)
    res = _dot(p, vp_ref[...])
    og = res[:, 0:_DH] / res[:, _DH:_DH + 1]
    o_ref[0, 0, 0:_BLK] = og[0:_BLK]
    o_ref[0, 0, _S - _BLK:_S] = og[_BLK:]

    def qk_group(base, ng):
        qq = q_ref[0, 0, pl.ds((base + 1) * _BLK, ng * _BLK)]
        qq = (qq * _SCALE).astype(jnp.bfloat16)
        mxg = row_bound(qq)
        sg = _dot_t(qq, kg_ref[...])
        states = []
        for jp in range(0, ng, 2):
            swp = _dot_t(
                qq[jp * _BLK:(jp + 2) * _BLK],
                kb_ref[pl.ds((base + jp) * _BLK, 4 * _BLK)],
            )
            for t in range(2):
                j = jp + t
                m = base + j
                r0 = rand_ref[h, m, 0]
                r1 = rand_ref[h, m, 1]
                r2 = rand_ref[h, m, 2]
                qmj = qq[j * _BLK:(j + 1) * _BLK]
                mx = mxg[j * _BLK:(j + 1) * _BLK]
                s_w = swp[t * _BLK:(t + 1) * _BLK, t * _BLK:t * _BLK + 3 * _BLK]
                s_0 = _dot_t(qmj, kb_ref[pl.ds(r0 * _BLK, _BLK)])
                s_1 = _dot_t(qmj, kb_ref[pl.ds(r1 * _BLK, _BLK)])
                s_2 = _dot_t(qmj, kb_ref[pl.ds(r2 * _BLK, _BLK)])
                s_g = sg[j * _BLK:(j + 1) * _BLK]
                es = [jnp.exp(sp - mx).astype(jnp.bfloat16)
                      for sp in (s_g, s_w, s_0, s_1, s_2)]
                states.append((es, r0, r1, r2))
        return states

    def pv_group(base, states):
        for jp in range(0, len(states), 2):
            eg2 = jnp.concatenate(
                [states[jp][0][0], states[jp + 1][0][0]], axis=0
            )
            accg2 = _dot(eg2, vg_ref[...])
            for t in range(2):
                j = jp + t
                m = base + j
                es, r0, r1, r2 = states[j]
                acc = accg2[t * _BLK:(t + 1) * _BLK]
                acc = acc + _dot(es[1], vp_ref[pl.ds(m * _BLK, 3 * _BLK)])
                acc = acc + _dot(es[2], vp_ref[pl.ds(r0 * _BLK, _BLK)])
                acc = acc + _dot(es[3], vp_ref[pl.ds(r1 * _BLK, _BLK)])
                acc = acc + _dot(es[4], vp_ref[pl.ds(r2 * _BLK, _BLK)])
                o_ref[0, 0, pl.ds((m + 1) * _BLK, _BLK)] = (
                    acc[:, 0:_DH] / acc[:, _DH:_DH + 1]
                )

    def body(i, carry):
        base = 6 * i
        pv_group(base, qk_group(base, 6))
        return carry

    jax.lax.fori_loop(0, 10, body, 0)
    pv_group(60, qk_group(60, 2))


def kernel(q, k, v, rand_attn):
    rand = rand_attn.astype(jnp.int32)

    def _spec(b, h, rand_ref):
        return (b, h, 0, 0)

    qkv_spec = pl.BlockSpec((1, 1, _S, _DH), _spec)
    out = pl.pallas_call(
        _bigbird_kernel,
        grid_spec=pltpu.PrefetchScalarGridSpec(
            num_scalar_prefetch=1,
            grid=(_B, _H),
            in_specs=[qkv_spec, qkv_spec, qkv_spec],
            out_specs=qkv_spec,
            scratch_shapes=[
                pltpu.VMEM((_S, _DH), jnp.bfloat16),
                pltpu.VMEM((_S, _VP), jnp.bfloat16),
                pltpu.VMEM((2 * _BLK, _DH), jnp.bfloat16),
                pltpu.VMEM((2 * _BLK, _VP), jnp.bfloat16),
            ],
        ),
        out_shape=jax.ShapeDtypeStruct((_B, _H, _S, _DH), jnp.float32),
        compiler_params=pltpu.CompilerParams(
            dimension_semantics=("parallel", "parallel"),
        ),
    )(rand, q, k, v)
    return out


# speedup vs baseline: 1.3077x; 1.1680x over previous
import numpy as np
import jax
import jax.numpy as jnp
from jax.experimental import pallas as pl
from jax.experimental.pallas import tpu as pltpu

_B, _H, _S, _DH, _BLK = 2, 16, 4096, 64, 64
_NB = _S // _BLK
_R = 3
_M = _NB - 2
_VP = 128
_SCALE = 1.0 / np.sqrt(_DH)
_BOUND_PAD = 1.01


def _dot_t(a, b):
    return jax.lax.dot_general(
        a, b, (((1,), (1,)), ((), ())), preferred_element_type=jnp.float32
    )


def _dot(a, b):
    return jnp.dot(a, b, preferred_element_type=jnp.float32)


def _bigbird_kernel(rand_ref, q_ref, k_ref, v_ref, o_ref,
                    kb_ref, vp_ref, kg_ref, vg_ref):
    h = pl.program_id(1)

    kb_ref[...] = k_ref[0, 0].astype(jnp.bfloat16)
    vp_ref[:, 0:_DH] = v_ref[0, 0].astype(jnp.bfloat16)
    vp_ref[:, _DH:_VP] = jnp.zeros((_S, _VP - _DH), jnp.bfloat16)
    vp_ref[:, _DH:_DH + 1] = jnp.ones((_S, 1), jnp.bfloat16)
    kg_ref[0:_BLK] = kb_ref[0:_BLK]
    kg_ref[_BLK:2 * _BLK] = kb_ref[_S - _BLK:_S]
    vg_ref[0:_BLK] = vp_ref[0:_BLK]
    vg_ref[_BLK:2 * _BLK] = vp_ref[_S - _BLK:_S]

    kf = k_ref[0, 0]
    kmax = jnp.sqrt(jnp.max(jnp.sum(kf * kf, axis=-1))) * _BOUND_PAD

    def row_bound(q_rows):
        qf = q_rows.astype(jnp.float32)
        return jnp.sqrt(
            jnp.sum(qf * qf, axis=-1, keepdims=True)
        ) * (kmax * _BOUND_PAD)

    qg = jnp.concatenate(
        [q_ref[0, 0, 0:_BLK], q_ref[0, 0, _S - _BLK:_S]], axis=0
    )
    qg = (qg * _SCALE).astype(jnp.bfloat16)
    mx_g = row_bound(qg)
    s = _dot_t(qg, kb_ref[...])
    p = jnp.exp(s - mx_g).astype(jnp.bfloat16)
    res = _dot(p, vp_ref[...])
    og = res[:, 0:_DH] / res[:, _DH:_DH + 1]
    o_ref[0, 0, 0:_BLK] = og[0:_BLK]
    o_ref[0, 0, _S - _BLK:_S] = og[_BLK:]

    def qk_group(base, ng):
        qq = q_ref[0, 0, pl.ds((base + 1) * _BLK, ng * _BLK)]
        qq = (qq * _SCALE).astype(jnp.bfloat16)
        mxg = row_bound(qq)
        sg = _dot_t(qq, kg_ref[...])
        states = []
        for jp in range(0, ng, 2):
            swp = _dot_t(
                qq[jp * _BLK:(jp + 2) * _BLK],
                kb_ref[pl.ds((base + jp) * _BLK, 4 * _BLK)],
            )
            for t in range(2):
                j = jp + t
                m = base + j
                r0 = rand_ref[h, m, 0]
                r1 = rand_ref[h, m, 1]
                r2 = rand_ref[h, m, 2]
                qmj = qq[j * _BLK:(j + 1) * _BLK]
                mx = mxg[j * _BLK:(j + 1) * _BLK]
                s_w = swp[t * _BLK:(t + 1) * _BLK, t * _BLK:t * _BLK + 3 * _BLK]
                s_0 = _dot_t(qmj, kb_ref[pl.ds(r0 * _BLK, _BLK)])
                s_1 = _dot_t(qmj, kb_ref[pl.ds(r1 * _BLK, _BLK)])
                s_2 = _dot_t(qmj, kb_ref[pl.ds(r2 * _BLK, _BLK)])
                s_g = sg[j * _BLK:(j + 1) * _BLK]
                es = [jnp.exp(sp - mx).astype(jnp.bfloat16)
                      for sp in (s_g, s_w, s_0, s_1, s_2)]
                states.append((es, r0, r1, r2))
        return states

    def pv_group(base, states):
        for jp in range(0, len(states), 2):
            eg2 = jnp.concatenate(
                [states[jp][0][0], states[jp + 1][0][0]], axis=0
            )
            accg2 = _dot(eg2, vg_ref[...])
            for t in range(2):
                j = jp + t
                m = base + j
                es, r0, r1, r2 = states[j]
                acc = accg2[t * _BLK:(t + 1) * _BLK]
                acc = acc + _dot(es[1], vp_ref[pl.ds(m * _BLK, 3 * _BLK)])
                acc = acc + _dot(es[2], vp_ref[pl.ds(r0 * _BLK, _BLK)])
                acc = acc + _dot(es[3], vp_ref[pl.ds(r1 * _BLK, _BLK)])
                acc = acc + _dot(es[4], vp_ref[pl.ds(r2 * _BLK, _BLK)])
                o_ref[0, 0, pl.ds((m + 1) * _BLK, _BLK)] = (
                    acc[:, 0:_DH] / acc[:, _DH:_DH + 1]
                )

    for base in range(0, 60, 6):
        pv_group(base, qk_group(base, 6))
    pv_group(60, qk_group(60, 2))


def kernel(q, k, v, rand_attn):
    rand = rand_attn.astype(jnp.int32)

    def _spec(b, h, rand_ref):
        return (b, h, 0, 0)

    qkv_spec = pl.BlockSpec((1, 1, _S, _DH), _spec)
    out = pl.pallas_call(
        _bigbird_kernel,
        grid_spec=pltpu.PrefetchScalarGridSpec(
            num_scalar_prefetch=1,
            grid=(_B, _H),
            in_specs=[qkv_spec, qkv_spec, qkv_spec],
            out_specs=qkv_spec,
            scratch_shapes=[
                pltpu.VMEM((_S, _DH), jnp.bfloat16),
                pltpu.VMEM((_S, _VP), jnp.bfloat16),
                pltpu.VMEM((2 * _BLK, _DH), jnp.bfloat16),
                pltpu.VMEM((2 * _BLK, _VP), jnp.bfloat16),
            ],
        ),
        out_shape=jax.ShapeDtypeStruct((_B, _H, _S, _DH), jnp.float32),
        compiler_params=pltpu.CompilerParams(
            dimension_semantics=("parallel", "parallel"),
        ),
    )(rand, q, k, v)
    return out
